# Initial kernel scaffold; baseline (speedup 1.0000x reference)
import jax
import jax.numpy as jnp
from jax.experimental import pallas as pl


def kernel(x, W0, W1, W2, W3):
    raise NotImplementedError("write your pallas kernel here")


# speedup vs baseline: 3.4154x; 3.4154x over previous
import functools

import jax
import jax.numpy as jnp
from jax import lax
from jax.experimental import pallas as pl
from jax.experimental.pallas import tpu as pltpu
from jax.experimental.pallas import tpu_sc as plsc

VOCAB = 100000
D = 32
NC, NS = 2, 16
NW = NC * NS
B = 16384 * 50
IDXW = 128
CHUNK = 1024
SUB = CHUNK // IDXW
PER_W = B // NW
N_CHUNKS = PER_W // CHUNK
ROWS2D = B // IDXW


def _body(x_ref, w0, w1, w2, w3, o0, o1, o2, o3, idx_v, rows_v, gsem, ssem):
    wid = lax.axis_index("s") * NC + lax.axis_index("c")
    tables = (w0, w1, w2, w3)
    outs = (o0, o1, o2, o3)

    def chunk_body(c, carry):
        rbase = wid * (PER_W // IDXW) + c * SUB
        obase = wid * PER_W + c * CHUNK
        pltpu.sync_copy(x_ref.at[pl.ds(rbase, SUB)], idx_v)
        for t in range(4):
            copies = []
            for j in range(SUB):
                cp = pltpu.async_copy(
                    tables[t].at[idx_v.at[j]],
                    rows_v.at[pl.ds(j * IDXW, IDXW)],
                    gsem,
                )
                copies.append(cp)
            for cp in copies:
                cp.wait()
            pltpu.sync_copy(rows_v, outs[t].at[pl.ds(obase, CHUNK)])
        return carry

    lax.fori_loop(0, N_CHUNKS, chunk_body, 0)


@jax.jit
def _sc_gather(x2d, W0, W1, W2, W3):
    f = pl.kernel(
        _body,
        out_type=[jax.ShapeDtypeStruct((B, D), jnp.float32)] * 4,
        mesh=plsc.VectorSubcoreMesh(core_axis_name="c", subcore_axis_name="s"),
        scratch_types=[
            pltpu.VMEM((SUB, IDXW), jnp.int32),
            pltpu.VMEM((CHUNK, D), jnp.float32),
            pltpu.SemaphoreType.DMA,
            pltpu.SemaphoreType.DMA,
        ],
        compiler_params=pltpu.CompilerParams(use_tc_tiling_on_sc=False),
    )
    return f(x2d, W0, W1, W2, W3)


def kernel(x, W0, W1, W2, W3):
    x2d = x.reshape(ROWS2D, IDXW).astype(jnp.int32)
    outs = _sc_gather(x2d, W0, W1, W2, W3)
    return tuple(o.reshape(16384, 50, D) for o in outs)


# speedup vs baseline: 3.4779x; 1.0183x over previous
import jax
import jax.numpy as jnp
from jax import lax
from jax.experimental import pallas as pl
from jax.experimental.pallas import tpu as pltpu
from jax.experimental.pallas import tpu_sc as plsc

VOCAB = 100000
D = 32
NC, NS = 2, 16
NW = NC * NS
B = 16384 * 50
IDXW = 128
CHUNK = 1024
SUB = CHUNK // IDXW
PER_W = B // NW
N_CHUNKS = PER_W // CHUNK
ROWS2D = B // IDXW


def _body(x_ref, w0, w1, w2, w3, o0, o1, o2, o3, idx_v, rows_v, gsem, ssem, isem):
    wid = lax.axis_index("s") * NC + lax.axis_index("c")
    tables = (w0, w1, w2, w3)
    outs = (o0, o1, o2, o3)
    rb0 = wid * (PER_W // IDXW)
    ob0 = wid * PER_W

    def fire_gathers(t, ib, b):
        for j in range(SUB):
            pltpu.async_copy(
                tables[t].at[idx_v.at[ib].at[j]],
                rows_v.at[b].at[pl.ds(j * IDXW, IDXW)],
                gsem,
            )

    def wait_gathers(b):
        for j in range(SUB):
            pltpu.make_async_copy(
                w0.at[pl.ds(0, IDXW)],
                rows_v.at[b].at[pl.ds(j * IDXW, IDXW)],
                gsem,
            ).wait()

    def fire_store(t, c, b):
        pltpu.async_copy(rows_v.at[b], outs[t].at[pl.ds(ob0 + c * CHUNK, CHUNK)], ssem)

    def wait_store(b):
        pltpu.make_async_copy(rows_v.at[b], o0.at[pl.ds(0, CHUNK)], ssem).wait()

    def wait_idx(ib):
        pltpu.make_async_copy(x_ref.at[pl.ds(0, SUB)], idx_v.at[ib], isem).wait()

    pltpu.sync_copy(x_ref.at[pl.ds(rb0, SUB)], idx_v.at[0])
    pltpu.async_copy(x_ref.at[pl.ds(rb0 + SUB, SUB)], idx_v.at[1], isem)
    fire_gathers(0, 0, 0)
    fire_gathers(1, 0, 1)

    def chunk_body(c, carry):
        ib = lax.rem(c, 2)
        nib = lax.rem(c + 1, 2)
        for t in range(4):
            b = t % 2
            wait_gathers(b)
            fire_store(t, c, b)
            if t < 2:
                wait_store(b)
                fire_gathers(t + 2, ib, b)
            elif t == 2:
                @pl.when(c < N_CHUNKS - 1)
                def _():
                    wait_idx(nib)
                    wait_store(b)
                    fire_gathers(0, nib, b)
            else:
                @pl.when(c < N_CHUNKS - 2)
                def _():
                    pltpu.async_copy(
                        x_ref.at[pl.ds(rb0 + (c + 2) * SUB, SUB)],
                        idx_v.at[ib],
                        isem,
                    )

                @pl.when(c < N_CHUNKS - 1)
                def _():
                    wait_store(b)
                    fire_gathers(1, nib, b)
        return carry

    lax.fori_loop(0, N_CHUNKS, chunk_body, 0)
    wait_store(0)
    wait_store(1)


@jax.jit
def _sc_gather(x2d, W0, W1, W2, W3):
    f = pl.kernel(
        _body,
        out_type=[jax.ShapeDtypeStruct((B, D), jnp.float32)] * 4,
        mesh=plsc.VectorSubcoreMesh(core_axis_name="c", subcore_axis_name="s"),
        scratch_types=[
            pltpu.VMEM((2, SUB, IDXW), jnp.int32),
            pltpu.VMEM((2, CHUNK, D), jnp.float32),
            pltpu.SemaphoreType.DMA,
            pltpu.SemaphoreType.DMA,
            pltpu.SemaphoreType.DMA,
        ],
        compiler_params=pltpu.CompilerParams(use_tc_tiling_on_sc=False),
    )
    return f(x2d, W0, W1, W2, W3)


def kernel(x, W0, W1, W2, W3):
    x2d = x.reshape(ROWS2D, IDXW).astype(jnp.int32)
    outs = _sc_gather(x2d, W0, W1, W2, W3)
    return tuple(o.reshape(16384, 50, D) for o in outs)


# speedup vs baseline: 3.4797x; 1.0005x over previous
import jax
import jax.numpy as jnp
from jax import lax
from jax.experimental import pallas as pl
from jax.experimental.pallas import tpu as pltpu
from jax.experimental.pallas import tpu_sc as plsc

VOCAB = 100000
D = 32
NC, NS = 2, 16
NW = NC * NS
B = 16384 * 50
CHUNK = 1600
PER_W = B // NW
N_CHUNKS = PER_W // CHUNK


def _body(x_ref, w0, w1, w2, w3, o0, o1, o2, o3, idx_v, rows_v, gsem, ssem, isem):
    wid = lax.axis_index("s") * NC + lax.axis_index("c")
    tables = (w0, w1, w2, w3)
    outs = (o0, o1, o2, o3)
    ob0 = wid * PER_W

    def fire_gather(t, ib, b):
        pltpu.async_copy(tables[t].at[idx_v.at[ib]], rows_v.at[b], gsem)

    def wait_gather(b):
        pltpu.make_async_copy(w0.at[pl.ds(0, CHUNK)], rows_v.at[b], gsem).wait()

    def fire_store(t, c, b):
        pltpu.async_copy(rows_v.at[b], outs[t].at[pl.ds(ob0 + c * CHUNK, CHUNK)], ssem)

    def wait_store(b):
        pltpu.make_async_copy(rows_v.at[b], o0.at[pl.ds(0, CHUNK)], ssem).wait()

    def wait_idx(ib):
        pltpu.make_async_copy(x_ref.at[pl.ds(0, CHUNK)], idx_v.at[ib], isem).wait()

    pltpu.sync_copy(x_ref.at[pl.ds(ob0, CHUNK)], idx_v.at[0])
    pltpu.async_copy(x_ref.at[pl.ds(ob0 + CHUNK, CHUNK)], idx_v.at[1], isem)
    fire_gather(0, 0, 0)
    fire_gather(1, 0, 1)

    def chunk_body(c, carry):
        ib = lax.rem(c, 2)
        nib = lax.rem(c + 1, 2)
        for t in range(4):
            b = t % 2
            wait_gather(b)
            fire_store(t, c, b)
            if t < 2:
                wait_store(b)
                fire_gather(t + 2, ib, b)
            elif t == 2:
                @pl.when(c < N_CHUNKS - 1)
                def _():
                    wait_idx(nib)
                    wait_store(b)
                    fire_gather(0, nib, b)
            else:
                @pl.when(c < N_CHUNKS - 2)
                def _():
                    pltpu.async_copy(
                        x_ref.at[pl.ds(ob0 + (c + 2) * CHUNK, CHUNK)],
                        idx_v.at[ib],
                        isem,
                    )

                @pl.when(c < N_CHUNKS - 1)
                def _():
                    wait_store(b)
                    fire_gather(1, nib, b)
        return carry

    lax.fori_loop(0, N_CHUNKS, chunk_body, 0)
    wait_store(0)
    wait_store(1)


@jax.jit
def _sc_gather(x1d, W0, W1, W2, W3):
    f = pl.kernel(
        _body,
        out_type=[jax.ShapeDtypeStruct((B, D), jnp.float32)] * 4,
        mesh=plsc.VectorSubcoreMesh(core_axis_name="c", subcore_axis_name="s"),
        scratch_types=[
            pltpu.VMEM((2, CHUNK), jnp.int32),
            pltpu.VMEM((2, CHUNK, D), jnp.float32),
            pltpu.SemaphoreType.DMA,
            pltpu.SemaphoreType.DMA,
            pltpu.SemaphoreType.DMA,
        ],
        compiler_params=pltpu.CompilerParams(use_tc_tiling_on_sc=False),
    )
    return f(x1d, W0, W1, W2, W3)


def kernel(x, W0, W1, W2, W3):
    x1d = x.reshape(B).astype(jnp.int32)
    outs = _sc_gather(x1d, W0, W1, W2, W3)
    return tuple(o.reshape(16384, 50, D) for o in outs)


# speedup vs baseline: 5.5051x; 1.5821x over previous
import jax
import jax.numpy as jnp
from jax import lax
from jax.experimental import pallas as pl
from jax.experimental.pallas import tpu as pltpu
from jax.experimental.pallas import tpu_sc as plsc

VOCAB = 100000
D = 32
NC, NS = 2, 16
NW = NC * NS
BATCH = 16384
HIST = 50
BW = BATCH // NW
NBT = BW // 128
NDT = D // 8
NSTEP = HIST * 4


def _body(xt_ref, w0, w1, w2, w3, o0, o1, o2, o3, xv, gbuf, tbuf, gsem, ssem):
    wid = lax.axis_index("s") * NC + lax.axis_index("c")
    tables = (w0, w1, w2, w3)
    outs = (o0, o1, o2, o3)
    b0 = wid * BW
    iota = jnp.arange(16, dtype=jnp.int32)

    def fire_gather(h, t, b):
        for k in range(4):
            @pl.when(t == k)
            def _():
                pltpu.async_copy(tables[k].at[xv.at[h]], gbuf.at[b], gsem)

    def wait_gather():
        pltpu.make_async_copy(w0.at[pl.ds(0, BW)], gbuf.at[0], gsem).wait()

    def fire_stores(h, t, b):
        for k in range(4):
            @pl.when(t == k)
            def _():
                for dt in range(NDT):
                    pltpu.async_copy(
                        tbuf.at[b].at[dt],
                        outs[k].at[h].at[dt].at[pl.ds(wid * NBT, NBT)],
                        ssem,
                    )

    def wait_stores():
        for dt in range(NDT):
            pltpu.make_async_copy(
                tbuf.at[0].at[dt], o0.at[0].at[dt].at[pl.ds(0, NBT)], ssem
            ).wait()

    def transpose(b):
        def m_body(m, carry):
            bt = m // 8
            k = lax.rem(m, 8)
            row_idx = iota + (bt * 128 + k * 16)
            bsplat = jnp.full((16,), b, jnp.int32)
            for dt in range(NDT):
                for d8 in range(8):
                    col = jnp.full((16,), dt * 8 + d8, jnp.int32)
                    v = plsc.load_gather(gbuf, [bsplat, row_idx, col])
                    tbuf[b, dt, bt, d8, pl.ds(k * 16, 16)] = v
            return carry

        lax.fori_loop(0, NBT * 8, m_body, 0)

    pltpu.sync_copy(xt_ref.at[:, pl.ds(b0, BW)], xv)
    fire_gather(0, 0, 0)

    def step(i, carry):
        h = i // 4
        t = lax.rem(i, 4)
        b = lax.rem(i, 2)
        ni = i + 1

        @pl.when(ni < NSTEP)
        def _():
            fire_gather(ni // 4, lax.rem(ni, 4), lax.rem(ni, 2))

        wait_gather()

        @pl.when(i >= 2)
        def _():
            wait_stores()

        transpose(b)
        fire_stores(h, t, b)
        return carry

    lax.fori_loop(0, NSTEP, step, 0)
    wait_stores()
    wait_stores()


@jax.jit
def _sc_gather(xt, W0, W1, W2, W3):
    f = pl.kernel(
        _body,
        out_type=[
            jax.ShapeDtypeStruct((HIST, NDT, BATCH // 128, 8, 128), jnp.float32)
        ] * 4,
        mesh=plsc.VectorSubcoreMesh(core_axis_name="c", subcore_axis_name="s"),
        scratch_types=[
            pltpu.VMEM((HIST, BW), jnp.int32),
            pltpu.VMEM((2, BW, D), jnp.float32),
            pltpu.VMEM((2, NDT, NBT, 8, 128), jnp.float32),
            pltpu.SemaphoreType.DMA,
            pltpu.SemaphoreType.DMA,
        ],
        compiler_params=pltpu.CompilerParams(
            use_tc_tiling_on_sc=False, needs_layout_passes=False
        ),
    )
    return f(xt, W0, W1, W2, W3)


def kernel(x, W0, W1, W2, W3):
    xt = x.T.astype(jnp.int32)
    outs = _sc_gather(xt, W0, W1, W2, W3)
    return tuple(
        o.transpose(2, 4, 0, 1, 3).reshape(BATCH, HIST, D) for o in outs
    )


# speedup vs baseline: 6.2778x; 1.1404x over previous
import jax
import jax.numpy as jnp
import numpy as np
from jax import lax
from jax.experimental import pallas as pl
from jax.experimental.pallas import tpu as pltpu
from jax.experimental.pallas import tpu_sc as plsc

VOCAB = 100000
D = 32
NC, NS = 2, 16
NW = NC * NS
BATCH = 16384
HIST = 50
BW = BATCH // NW
NBT = BW // 128
NDT = D // 8
NSTEP = HIST * 4
TBLK = NDT * 8 * 128


def _body(xt_ref, w0, w1, w2, w3, o0, o1, o2, o3, xv, gbuf, tbuf, cvec, gsem, ssem):
    wid = lax.axis_index("s") * NC + lax.axis_index("c")
    tables = (w0, w1, w2, w3)
    outs = (o0, o1, o2, o3)
    b0 = wid * BW
    iota = lax.iota(jnp.int32, 16)

    for j in range(16):
        rot = (iota + j) & 15
        cvec[0, j, :] = rot
        cvec[1, j, :] = ((rot >> 3) << 12) + ((rot & 7) << 7) + iota

    def fire_gather(h, t, b):
        for k in range(4):
            @pl.when(t == k)
            def _():
                pltpu.async_copy(
                    tables[k].at[xv.at[h]], gbuf.at[pl.ds(b * BW, BW)], gsem
                )

    def wait_gather():
        pltpu.make_async_copy(
            w0.at[pl.ds(0, BW)], gbuf.at[pl.ds(0, BW)], gsem
        ).wait()

    def fire_stores(h, t, b):
        for k in range(4):
            @pl.when(t == k)
            def _():
                for dt in range(NDT):
                    pltpu.async_copy(
                        tbuf.at[pl.ds(b * 4 * TBLK + dt * TBLK, TBLK)],
                        outs[k].at[h].at[dt].at[pl.ds(wid * TBLK, TBLK)],
                        ssem,
                    )

    def wait_stores():
        for dt in range(NDT):
            pltpu.make_async_copy(
                tbuf.at[pl.ds(dt * TBLK, TBLK)],
                o0.at[0].at[0].at[pl.ds(0, TBLK)],
                ssem,
            ).wait()

    def transpose(b):
        for c in range(2):

            def m_body(m, carry):
                bt = m // 8
                kb = lax.rem(m, 8)
                rowvec = (b * BW + bt * 128 + kb * 16) + iota
                baseo = b * 16384 + c * 8192 + bt * 1024 + kb * 16
                for j in range(16):
                    col = cvec[0, j, :] + (c * 16)
                    pos = cvec[1, j, :] + baseo
                    v = plsc.load_gather(gbuf, [rowvec, col])
                    plsc.store_scatter(tbuf, [pos], v)
                return carry

            lax.fori_loop(0, NBT * 8, m_body, 0)

    pltpu.sync_copy(xt_ref.at[:, pl.ds(b0, BW)], xv)
    fire_gather(0, 0, 0)

    def step(i, carry):
        h = i // 4
        t = lax.rem(i, 4)
        b = lax.rem(i, 2)
        ni = i + 1

        @pl.when(ni < NSTEP)
        def _():
            fire_gather(ni // 4, lax.rem(ni, 4), lax.rem(ni, 2))

        wait_gather()

        @pl.when(i >= 2)
        def _():
            wait_stores()

        transpose(b)
        fire_stores(h, t, b)
        return carry

    lax.fori_loop(0, NSTEP, step, 0)
    wait_stores()
    wait_stores()


@jax.jit
def _sc_gather(xt, W0, W1, W2, W3):
    f = pl.kernel(
        _body,
        out_type=[
            jax.ShapeDtypeStruct((HIST, NDT, (BATCH // 128) * 8 * 128), jnp.float32)
        ] * 4,
        mesh=plsc.VectorSubcoreMesh(core_axis_name="c", subcore_axis_name="s"),
        scratch_types=[
            pltpu.VMEM((HIST, BW), jnp.int32),
            pltpu.VMEM((2 * BW, D), jnp.float32),
            pltpu.VMEM((2 * 4 * TBLK,), jnp.float32),
            pltpu.VMEM((2, 16, 16), jnp.int32),
            pltpu.SemaphoreType.DMA,
            pltpu.SemaphoreType.DMA,
        ],
        compiler_params=pltpu.CompilerParams(
            use_tc_tiling_on_sc=False, needs_layout_passes=False
        ),
    )
    return f(xt, W0, W1, W2, W3)


def kernel(x, W0, W1, W2, W3):
    xt = x.T.astype(jnp.int32)
    outs = _sc_gather(xt, W0, W1, W2, W3)
    return tuple(
        o.reshape(HIST, NDT, BATCH // 128, 8, 128)
        .transpose(2, 4, 0, 1, 3)
        .reshape(BATCH, HIST, D)
        for o in outs
    )
